# baseline (speedup 1.0000x reference)
import jax
import jax.numpy as jnp
from jax import lax
from jax.experimental import pallas as pl
from jax.experimental.pallas import tpu as pltpu
from jax.experimental.pallas import tpu_sc as plsc

B, H, W = 4, 224, 224
C = 128
NROW = 224
L = 16
NPOS = H * W
NPB = 8
P = NPOS // NPB
CB = 16
PC = 896
NPC = P // PC
NCB_HALF = C // CB


def _body(i_hbm, j_hbm, ti_hbm, tj_hbm, out_hbm,
          ti_v, tj_v, idx_i, idx_j, outbuf, sem0, sem1):
    info = plsc.get_sparse_core_info()
    nc, ns = info.num_cores, info.num_subcores

    wid = lax.axis_index("s") * nc + lax.axis_index("c")
    b = wid // NPB
    p0 = (wid % NPB) * P

    pltpu.sync_copy(ti_hbm, ti_v)
    pltpu.sync_copy(tj_hbm, tj_v)
    pltpu.sync_copy(i_hbm.at[pl.ds(b * NPOS + p0, P)], idx_i)
    pltpu.sync_copy(j_hbm.at[pl.ds(b * NPOS + p0, P)], idx_j)

    sems = (sem0, sem1)

    def fill_slab(table_v, idx_v, cbase, pc, buf):
        @plsc.parallel_loop(0, PC // L, 1, unroll=2)
        def pb_loop(pb):
            q = pb * L
            iv = idx_v[pl.ds(pc * PC + q, L)] * C + cbase
            for c in range(CB):
                v = plsc.load_gather(table_v, [iv + c])
                outbuf[buf, c, pl.ds(q, L)] = v

    def out_dma(row0, poff, buf, sem):
        return pltpu.make_async_copy(
            outbuf.at[buf],
            out_hbm.at[pl.ds(row0, CB), pl.ds(poff, PC)],
            sem,
        )

    for half, (table_v, idx_v) in enumerate(((ti_v, idx_i), (tj_v, idx_j))):
        def cb_body(cbi, _, table_v=table_v, idx_v=idx_v, half=half):
            cbase = cbi * CB
            row0 = b * 2 * C + half * C + cbase
            for pc in range(NPC):
                buf = pc & 1

                def wait_prev():
                    out_dma(row0, p0, buf, sems[buf]).wait()

                if half == 0 and pc < 2:
                    @pl.when(cbi >= 1)
                    def _():
                        wait_prev()
                else:
                    wait_prev()
                fill_slab(table_v, idx_v, cbase, pc, buf)
                out_dma(row0, p0 + pc * PC, buf, sems[buf]).start()
            return 0

        lax.fori_loop(0, NCB_HALF, cb_body, 0)

    out_dma(0, 0, 0, sem0).wait()
    out_dma(0, 0, 1, sem1).wait()


@jax.jit
def _position_embedding_sc(i, j, table_i, table_j):
    mesh = plsc.VectorSubcoreMesh(core_axis_name="c", subcore_axis_name="s")
    fn = pl.kernel(
        _body,
        out_type=jax.ShapeDtypeStruct((B * 2 * C, NPOS), jnp.float32),
        mesh=mesh,
        scratch_types=[
            pltpu.VMEM((NROW * C,), jnp.float32),
            pltpu.VMEM((NROW * C,), jnp.float32),
            pltpu.VMEM((P,), jnp.int32),
            pltpu.VMEM((P,), jnp.int32),
            pltpu.VMEM((2, CB, PC), jnp.float32),
            pltpu.SemaphoreType.DMA,
            pltpu.SemaphoreType.DMA,
        ],
        compiler_params=pltpu.CompilerParams(needs_layout_passes=False),
    )
    out2d = fn(i.reshape(-1), j.reshape(-1),
               table_i.reshape(-1), table_j.reshape(-1))
    return out2d.reshape(B, 2 * C, H, W)


def kernel(i, j, table_i, table_j):
    return _position_embedding_sc(i, j, table_i, table_j)


# speedup vs baseline: 2.0616x; 2.0616x over previous
import jax
import jax.numpy as jnp
from jax import lax
from jax.experimental import pallas as pl
from jax.experimental.pallas import tpu as pltpu
from jax.experimental.pallas import tpu_sc as plsc

B, H, W = 4, 224, 224
C = 128
NROW = 224
L = 16
NPOS = H * W
NPB = 8
P = NPOS // NPB
CB = 16
PC = 896
NPC = P // PC
NCB_HALF = C // CB


def _body(i_hbm, j_hbm, ti_hbm, tj_hbm, out_hbm,
          ti_v, tj_v, idx_i, idx_j, outbuf, sem0, sem1):
    info = plsc.get_sparse_core_info()
    nc, ns = info.num_cores, info.num_subcores

    wid = lax.axis_index("s") * nc + lax.axis_index("c")
    b = wid // NPB
    p0 = (wid % NPB) * P

    pltpu.sync_copy(ti_hbm, ti_v)
    pltpu.sync_copy(tj_hbm, tj_v)
    pltpu.sync_copy(i_hbm.at[pl.ds(b * NPOS + p0, P)], idx_i)
    pltpu.sync_copy(j_hbm.at[pl.ds(b * NPOS + p0, P)], idx_j)

    sems = (sem0, sem1)

    def fill_slab(table_v, idx_v, cbase, pc, buf):
        coff = cbase * NROW
        @plsc.parallel_loop(0, PC // L, 1, unroll=2)
        def pb_loop(pb):
            q = pb * L
            iv = idx_v[pl.ds(pc * PC + q, L)] + coff
            for c in range(CB):
                v = plsc.load_gather(table_v, [iv + c * NROW])
                outbuf[buf, c, pl.ds(q, L)] = v

    def out_dma(row0, poff, buf, sem):
        return pltpu.make_async_copy(
            outbuf.at[buf],
            out_hbm.at[pl.ds(row0, CB), pl.ds(poff, PC)],
            sem,
        )

    for half, (table_v, idx_v) in enumerate(((ti_v, idx_i), (tj_v, idx_j))):
        def cb_body(cbi, _, table_v=table_v, idx_v=idx_v, half=half):
            cbase = cbi * CB
            row0 = b * 2 * C + half * C + cbase
            for pc in range(NPC):
                buf = pc & 1

                def wait_prev():
                    out_dma(row0, p0, buf, sems[buf]).wait()

                if half == 0 and pc < 2:
                    @pl.when(cbi >= 1)
                    def _():
                        wait_prev()
                else:
                    wait_prev()
                fill_slab(table_v, idx_v, cbase, pc, buf)
                out_dma(row0, p0 + pc * PC, buf, sems[buf]).start()
            return 0

        lax.fori_loop(0, NCB_HALF, cb_body, 0)

    out_dma(0, 0, 0, sem0).wait()
    out_dma(0, 0, 1, sem1).wait()


@jax.jit
def _position_embedding_sc(i, j, table_i, table_j):
    mesh = plsc.VectorSubcoreMesh(core_axis_name="c", subcore_axis_name="s")
    fn = pl.kernel(
        _body,
        out_type=jax.ShapeDtypeStruct((B * 2 * C, NPOS), jnp.float32),
        mesh=mesh,
        scratch_types=[
            pltpu.VMEM((NROW * C,), jnp.float32),
            pltpu.VMEM((NROW * C,), jnp.float32),
            pltpu.VMEM((P,), jnp.int32),
            pltpu.VMEM((P,), jnp.int32),
            pltpu.VMEM((2, CB, PC), jnp.float32),
            pltpu.SemaphoreType.DMA,
            pltpu.SemaphoreType.DMA,
        ],
        compiler_params=pltpu.CompilerParams(needs_layout_passes=False),
    )
    out2d = fn(i.reshape(-1), j.reshape(-1),
               table_i.T.reshape(-1), table_j.T.reshape(-1))
    return out2d.reshape(B, 2 * C, H, W)


def kernel(i, j, table_i, table_j):
    return _position_embedding_sc(i, j, table_i, table_j)


# speedup vs baseline: 2.3557x; 1.1426x over previous
import jax
import jax.numpy as jnp
from jax import lax
from jax.experimental import pallas as pl
from jax.experimental.pallas import tpu as pltpu
from jax.experimental.pallas import tpu_sc as plsc

B, H, W = 4, 224, 224
C = 128
NROW = 224
L = 16
NWB = W // L
HB = 56
NHBLK = H // HB
CB = 16
HS = 8
NHS = HB // HS
NCB = C // CB


def _body(i_hbm, j_hbm, ti_hbm, tj_hbm, out_hbm,
          tab_v, idx_v, outbuf, sem0, sem1):
    info = plsc.get_sparse_core_info()
    nc, ns = info.num_cores, info.num_subcores

    wid = lax.axis_index("s") * nc + lax.axis_index("c")
    task = wid // 2
    half = wid % 2
    b = task // NHBLK
    h0 = (task % NHBLK) * HB
    chan0 = half * C

    @pl.when(half == 0)
    def _():
        pltpu.sync_copy(ti_hbm, tab_v)
        pltpu.sync_copy(i_hbm.at[b, pl.ds(h0, HB), :], idx_v)

    @pl.when(half == 1)
    def _():
        pltpu.sync_copy(tj_hbm, tab_v)
        pltpu.sync_copy(j_hbm.at[b, pl.ds(h0, HB), :], idx_v)

    sems = (sem0, sem1)

    def fill_slab(cbase, hs, buf):
        coff = cbase * NROW

        @plsc.parallel_loop(0, NWB, 1, unroll=1)
        def wb_loop(wb):
            woff = wb * L
            for hh in range(HS):
                iv = idx_v[hs * HS + hh, pl.ds(woff, L)] + coff
                for c in range(CB):
                    v = plsc.load_gather(tab_v, [iv + c * NROW])
                    outbuf[buf, c, hh, pl.ds(woff, L)] = v

    def out_dma(cbase, hs, buf, sem):
        return pltpu.make_async_copy(
            outbuf.at[buf],
            out_hbm.at[b, pl.ds(chan0 + cbase, CB), pl.ds(h0 + hs * HS, HS), :],
            sem,
        )

    def cb_body(cbi, _):
        cbase = cbi * CB

        def hs_pair(s, _):
            for k in range(2):
                hs = 2 * s + k
                @pl.when((cbi >= 1) | (s >= 1))
                def _():
                    out_dma(0, 0, k, sems[k]).wait()
                fill_slab(cbase, hs, k)
                out_dma(cbase, hs, k, sems[k]).start()
            return 0

        lax.fori_loop(0, NHS // 2, hs_pair, 0)

        out_dma(0, 0, 0, sem0).wait()
        fill_slab(cbase, NHS - 1, 0)
        out_dma(cbase, NHS - 1, 0, sem0).start()
        return 0

    lax.fori_loop(0, NCB, cb_body, 0)

    out_dma(0, 0, 0, sem0).wait()
    out_dma(0, 0, 1, sem1).wait()


@jax.jit
def _position_embedding_sc(i, j, table_i, table_j):
    mesh = plsc.VectorSubcoreMesh(core_axis_name="c", subcore_axis_name="s")
    fn = pl.kernel(
        _body,
        out_type=jax.ShapeDtypeStruct((B, 2 * C, H, W), jnp.float32),
        mesh=mesh,
        scratch_types=[
            pltpu.VMEM((NROW * C,), jnp.float32),
            pltpu.VMEM((HB, W), jnp.int32),
            pltpu.VMEM((2, CB, HS, W), jnp.float32),
            pltpu.SemaphoreType.DMA,
            pltpu.SemaphoreType.DMA,
        ],
        compiler_params=pltpu.CompilerParams(needs_layout_passes=False),
    )
    return fn(i, j, table_i.T.reshape(-1), table_j.T.reshape(-1))


def kernel(i, j, table_i, table_j):
    return _position_embedding_sc(i, j, table_i, table_j)


# speedup vs baseline: 2.3669x; 1.0048x over previous
import jax
import jax.numpy as jnp
from jax import lax
from jax.experimental import pallas as pl
from jax.experimental.pallas import tpu as pltpu
from jax.experimental.pallas import tpu_sc as plsc

B, H, W = 4, 224, 224
C = 128
NROW = 224
L = 16
NWB = W // L
HB = 56
NHBLK = H // HB
CB = 16
HS = 8
NHS = HB // HS
NCB = C // CB


def _body(i_hbm, j_hbm, ti_hbm, tj_hbm, out_hbm,
          tab_v, idx_v, outbuf, sem0, sem1):
    info = plsc.get_sparse_core_info()
    nc, ns = info.num_cores, info.num_subcores

    wid = lax.axis_index("s") * nc + lax.axis_index("c")
    task = wid // 2
    half = wid % 2
    b = task // NHBLK
    h0 = (task % NHBLK) * HB
    chan0 = half * C

    @pl.when(half == 0)
    def _():
        pltpu.sync_copy(ti_hbm, tab_v)
        pltpu.sync_copy(i_hbm.at[b, pl.ds(h0, HB), :], idx_v)

    @pl.when(half == 1)
    def _():
        pltpu.sync_copy(tj_hbm, tab_v)
        pltpu.sync_copy(j_hbm.at[b, pl.ds(h0, HB), :], idx_v)

    sems = (sem0, sem1)

    def fill_slab(cbase, hs, buf):
        coff = cbase * NROW

        @plsc.parallel_loop(0, NWB, 1, unroll=1)
        def wb_loop(wb):
            woff = wb * L
            for hh in range(HS):
                iv = idx_v[hs * HS + hh, pl.ds(woff, L)] + coff
                for c in range(CB):
                    v = plsc.load_gather(tab_v, [iv + c * NROW])
                    outbuf[buf, c, hh, pl.ds(woff, L)] = v

    def out_dma(cbase, hs, buf, sem):
        return pltpu.make_async_copy(
            outbuf.at[buf],
            out_hbm.at[b, pl.ds(chan0 + cbase, CB), pl.ds(h0 + hs * HS, HS), :],
            sem,
        )

    def cb_body(cbi, _):
        cbase = cbi * CB

        def hs_pair(s, _):
            for k in range(2):
                hs = 2 * s + k
                @pl.when((cbi >= 1) | (s >= 1))
                def _():
                    out_dma(0, 0, k, sems[k]).wait()
                fill_slab(cbase, hs, k)
                out_dma(cbase, hs, k, sems[k]).start()
            return 0

        lax.fori_loop(0, NHS // 2, hs_pair, 0)

        out_dma(0, 0, 0, sem0).wait()
        fill_slab(cbase, NHS - 1, 0)
        out_dma(cbase, NHS - 1, 0, sem0).start()
        return 0

    lax.fori_loop(0, NCB, cb_body, 0)

    out_dma(0, 0, 0, sem0).wait()
    out_dma(0, 0, 1, sem1).wait()


@jax.jit
def _position_embedding_sc(i, j, table_i, table_j):
    mesh = plsc.VectorSubcoreMesh(core_axis_name="c", subcore_axis_name="s")
    fn = pl.kernel(
        _body,
        out_type=jax.ShapeDtypeStruct((B, 2 * C, H, W), jnp.float32),
        mesh=mesh,
        scratch_types=[
            pltpu.VMEM((NROW * C,), jnp.float32),
            pltpu.VMEM((HB, W), jnp.int32),
            pltpu.VMEM((2, CB, HS, W), jnp.float32),
            pltpu.SemaphoreType.DMA,
            pltpu.SemaphoreType.DMA,
        ],
        compiler_params=pltpu.CompilerParams(
            needs_layout_passes=False, use_tc_tiling_on_sc=True
        ),
    )
    return fn(i, j, table_i.T.reshape(-1), table_j.T.reshape(-1))


def kernel(i, j, table_i, table_j):
    return _position_embedding_sc(i, j, table_i, table_j)


# speedup vs baseline: 2.9329x; 1.2391x over previous
import jax
import jax.numpy as jnp
from jax import lax
from jax.experimental import pallas as pl
from jax.experimental.pallas import tpu as pltpu
from jax.experimental.pallas import tpu_sc as plsc

B, H, W = 4, 224, 224
C = 128
NROW = 224
NPOS = H * W
NPB = 8
P = NPOS // NPB
N = 112
NCH = P // N


def _body(i_hbm, j_hbm, ti_hbm, tj_hbm, out_hbm,
          idx_i, idx_j, buf_i, buf_j,
          gsem_i0, gsem_i1, gsem_j0, gsem_j1, osem_i0, osem_i1,
          osem_j0, osem_j1):
    info = plsc.get_sparse_core_info()
    nc, ns = info.num_cores, info.num_subcores

    wid = lax.axis_index("s") * nc + lax.axis_index("c")
    b = wid // NPB
    p0 = (wid % NPB) * P

    pltpu.sync_copy(i_hbm.at[pl.ds(b * NPOS + p0, P)], idx_i)
    pltpu.sync_copy(j_hbm.at[pl.ds(b * NPOS + p0, P)], idx_j)

    gsems = ((gsem_i0, gsem_j0), (gsem_i1, gsem_j1))
    osems = ((osem_i0, osem_j0), (osem_i1, osem_j1))

    def gathers(ck, k):
        gi = pltpu.make_async_copy(
            ti_hbm.at[idx_i.at[pl.ds(ck * N, N)]], buf_i.at[k], gsems[k][0])
        gj = pltpu.make_async_copy(
            tj_hbm.at[idx_j.at[pl.ds(ck * N, N)]], buf_j.at[k], gsems[k][1])
        return gi, gj

    def out_dmas(ck, k):
        pos = p0 + ck * N
        oi = pltpu.make_async_copy(
            buf_i.at[k], out_hbm.at[b, pl.ds(pos, N), pl.ds(0, C)],
            osems[k][0])
        oj = pltpu.make_async_copy(
            buf_j.at[k], out_hbm.at[b, pl.ds(pos, N), pl.ds(C, C)],
            osems[k][1])
        return oi, oj

    def ck_pair(s, _):
        for k in range(2):
            ck = 2 * s + k

            @pl.when(s >= 1)
            def _():
                oi, oj = out_dmas(0, k)
                oi.wait()
                oj.wait()

            gi, gj = gathers(ck, k)
            gi.start()
            gj.start()
            gi.wait()
            gj.wait()
            oi, oj = out_dmas(ck, k)
            oi.start()
            oj.start()
        return 0

    lax.fori_loop(0, NCH // 2, ck_pair, 0)

    for k in range(2):
        oi, oj = out_dmas(0, k)
        oi.wait()
        oj.wait()


@jax.jit
def _position_embedding_sc(i, j, table_i, table_j):
    mesh = plsc.VectorSubcoreMesh(core_axis_name="c", subcore_axis_name="s")
    fn = pl.kernel(
        _body,
        out_type=jax.ShapeDtypeStruct((B, NPOS, 2 * C), jnp.float32),
        mesh=mesh,
        scratch_types=[
            pltpu.VMEM((P,), jnp.int32),
            pltpu.VMEM((P,), jnp.int32),
            pltpu.VMEM((2, N, C), jnp.float32),
            pltpu.VMEM((2, N, C), jnp.float32),
            pltpu.SemaphoreType.DMA,
            pltpu.SemaphoreType.DMA,
            pltpu.SemaphoreType.DMA,
            pltpu.SemaphoreType.DMA,
            pltpu.SemaphoreType.DMA,
            pltpu.SemaphoreType.DMA,
            pltpu.SemaphoreType.DMA,
            pltpu.SemaphoreType.DMA,
        ],
        compiler_params=pltpu.CompilerParams(needs_layout_passes=False),
    )
    outp = fn(i.reshape(-1), j.reshape(-1), table_i, table_j)
    return jnp.transpose(outp.reshape(B, H, W, 2 * C), (0, 3, 1, 2))


def kernel(i, j, table_i, table_j):
    return _position_embedding_sc(i, j, table_i, table_j)


# speedup vs baseline: 2.9367x; 1.0013x over previous
import jax
import jax.numpy as jnp
from jax import lax
from jax.experimental import pallas as pl
from jax.experimental.pallas import tpu as pltpu
from jax.experimental.pallas import tpu_sc as plsc

B, H, W = 4, 224, 224
C = 128
NROW = 224
NPOS = H * W
NPB = 8
P = NPOS // NPB
N = 112
NCH = P // N


NBUF = 4


def _body(i_hbm, j_hbm, ti_hbm, tj_hbm, out_hbm,
          idx_i, idx_j, buf_i, buf_j, gsem_i, gsem_j, osem_i, osem_j):
    info = plsc.get_sparse_core_info()
    nc, ns = info.num_cores, info.num_subcores

    wid = lax.axis_index("s") * nc + lax.axis_index("c")
    b = wid // NPB
    p0 = (wid % NPB) * P

    pltpu.sync_copy(i_hbm.at[pl.ds(b * NPOS + p0, P)], idx_i)
    pltpu.sync_copy(j_hbm.at[pl.ds(b * NPOS + p0, P)], idx_j)

    def gathers(ck, k):
        gi = pltpu.make_async_copy(
            ti_hbm.at[idx_i.at[pl.ds(ck * N, N)]], buf_i.at[k],
            gsem_i.at[k])
        gj = pltpu.make_async_copy(
            tj_hbm.at[idx_j.at[pl.ds(ck * N, N)]], buf_j.at[k],
            gsem_j.at[k])
        return gi, gj

    def out_dmas(ck, k):
        pos = p0 + ck * N
        oi = pltpu.make_async_copy(
            buf_i.at[k], out_hbm.at[b, pl.ds(pos, N), pl.ds(0, C)],
            osem_i.at[k])
        oj = pltpu.make_async_copy(
            buf_j.at[k], out_hbm.at[b, pl.ds(pos, N), pl.ds(C, C)],
            osem_j.at[k])
        return oi, oj

    for k in range(2):
        gi, gj = gathers(k, k)
        gi.start()
        gj.start()

    def ck_quad(s, _):
        for k in range(NBUF):
            ck = NBUF * s + k
            gi, gj = gathers(ck, k)
            gi.wait()
            gj.wait()
            oi, oj = out_dmas(ck, k)
            oi.start()
            oj.start()
            nxt = ck + 2
            tgt = (k + 2) % NBUF

            @pl.when((nxt >= NBUF) & (nxt < NCH))
            def _():
                po, pj = out_dmas(0, tgt)
                po.wait()
                pj.wait()

            @pl.when(nxt < NCH)
            def _():
                pgi, pgj = gathers(nxt, tgt)
                pgi.start()
                pgj.start()
        return 0

    lax.fori_loop(0, NCH // NBUF, ck_quad, 0)

    for k in range(NBUF):
        oi, oj = out_dmas(0, k)
        oi.wait()
        oj.wait()


@jax.jit
def _position_embedding_sc(i, j, table_i, table_j):
    mesh = plsc.VectorSubcoreMesh(core_axis_name="c", subcore_axis_name="s")
    fn = pl.kernel(
        _body,
        out_type=jax.ShapeDtypeStruct((B, NPOS, 2 * C), jnp.float32),
        mesh=mesh,
        scratch_types=[
            pltpu.VMEM((P,), jnp.int32),
            pltpu.VMEM((P,), jnp.int32),
            pltpu.VMEM((NBUF, N, C), jnp.float32),
            pltpu.VMEM((NBUF, N, C), jnp.float32),
            pltpu.SemaphoreType.DMA((NBUF,)),
            pltpu.SemaphoreType.DMA((NBUF,)),
            pltpu.SemaphoreType.DMA((NBUF,)),
            pltpu.SemaphoreType.DMA((NBUF,)),
        ],
        compiler_params=pltpu.CompilerParams(needs_layout_passes=False),
    )
    outp = fn(i.reshape(-1), j.reshape(-1), table_i, table_j)
    return jnp.transpose(outp.reshape(B, H, W, 2 * C), (0, 3, 1, 2))


def kernel(i, j, table_i, table_j):
    return _position_embedding_sc(i, j, table_i, table_j)


# speedup vs baseline: 10.3803x; 3.5347x over previous
import jax
import jax.numpy as jnp
from jax import lax
from jax.experimental import pallas as pl
from jax.experimental.pallas import tpu as pltpu
from jax.experimental.pallas import tpu_sc as plsc

B, H, W = 4, 224, 224
C = 128
NROW = 224
L = 16
NPOS = H * W
NPB = 8
P = NPOS // NPB
N = 64
NCH = P // N


def _body(i_hbm, j_hbm, ti_hbm, tj_hbm, out_hbm,
          ti_v, tj_v, idx_i, idx_j, outbuf, sem0, sem1):
    info = plsc.get_sparse_core_info()
    nc, ns = info.num_cores, info.num_subcores

    wid = lax.axis_index("s") * nc + lax.axis_index("c")
    b = wid // NPB
    p0 = (wid % NPB) * P

    pltpu.sync_copy(ti_hbm, ti_v)
    pltpu.sync_copy(tj_hbm, tj_v)
    pltpu.sync_copy(i_hbm.at[pl.ds(b * NPOS + p0, P)], idx_i)
    pltpu.sync_copy(j_hbm.at[pl.ds(b * NPOS + p0, P)], idx_j)

    sems = (sem0, sem1)
    lane = lax.broadcasted_iota(jnp.int32, (L,), 0)
    cvecs = [cb * L + lane for cb in range(C // L)]

    def fill_slab(ck, buf):
        @plsc.parallel_loop(0, N, 1, unroll=2)
        def pos_loop(p):
            pv = lane * 0 + (ck * N + p)
            bi = plsc.load_gather(idx_i, [pv]) * C
            bj = plsc.load_gather(idx_j, [pv]) * C
            for cb in range(C // L):
                v = plsc.load_gather(ti_v, [bi + cvecs[cb]])
                outbuf[buf, p, pl.ds(cb * L, L)] = v
                w = plsc.load_gather(tj_v, [bj + cvecs[cb]])
                outbuf[buf, p, pl.ds(C + cb * L, L)] = w

    def out_dma(ck, buf, sem):
        return pltpu.make_async_copy(
            outbuf.at[buf],
            out_hbm.at[b, pl.ds(p0 + ck * N, N), :],
            sem,
        )

    def ck_pair(s, _):
        for k in range(2):
            ck = 2 * s + k

            @pl.when(s >= 1)
            def _():
                out_dma(0, k, sems[k]).wait()

            fill_slab(ck, k)
            out_dma(ck, k, sems[k]).start()
        return 0

    lax.fori_loop(0, NCH // 2, ck_pair, 0)

    out_dma(0, 0, sem0).wait()
    out_dma(0, 1, sem1).wait()


@jax.jit
def _position_embedding_sc(i, j, table_i, table_j):
    mesh = plsc.VectorSubcoreMesh(core_axis_name="c", subcore_axis_name="s")
    fn = pl.kernel(
        _body,
        out_type=jax.ShapeDtypeStruct((B, NPOS, 2 * C), jnp.float32),
        mesh=mesh,
        scratch_types=[
            pltpu.VMEM((NROW * C,), jnp.float32),
            pltpu.VMEM((NROW * C,), jnp.float32),
            pltpu.VMEM((P,), jnp.int32),
            pltpu.VMEM((P,), jnp.int32),
            pltpu.VMEM((2, N, 2 * C), jnp.float32),
            pltpu.SemaphoreType.DMA,
            pltpu.SemaphoreType.DMA,
        ],
        compiler_params=pltpu.CompilerParams(needs_layout_passes=False),
    )
    outp = fn(i.reshape(-1), j.reshape(-1),
              table_i.reshape(-1), table_j.reshape(-1))
    return jnp.transpose(outp.reshape(B, H, W, 2 * C), (0, 3, 1, 2))


def kernel(i, j, table_i, table_j):
    return _position_embedding_sc(i, j, table_i, table_j)


# speedup vs baseline: 10.4621x; 1.0079x over previous
import jax
import jax.numpy as jnp
from jax import lax
from jax.experimental import pallas as pl
from jax.experimental.pallas import tpu as pltpu
from jax.experimental.pallas import tpu_sc as plsc

B, H, W = 4, 224, 224
C = 128
NROW = 224
L = 16
NPOS = H * W
NPB = 8
P = NPOS // NPB
N = 64
NCH = P // N


def _body(i_hbm, j_hbm, ti_hbm, tj_hbm, out_hbm,
          ti_v, tj_v, idx_i, idx_j, outbuf, sem0, sem1):
    info = plsc.get_sparse_core_info()
    nc, ns = info.num_cores, info.num_subcores

    wid = lax.axis_index("s") * nc + lax.axis_index("c")
    b = wid // NPB
    p0 = (wid % NPB) * P

    pltpu.sync_copy(ti_hbm, ti_v)
    pltpu.sync_copy(tj_hbm, tj_v)
    pltpu.sync_copy(i_hbm.at[pl.ds(b * NPOS + p0, P)], idx_i)
    pltpu.sync_copy(j_hbm.at[pl.ds(b * NPOS + p0, P)], idx_j)

    sems = (sem0, sem1)
    lane = lax.broadcasted_iota(jnp.int32, (L,), 0)
    cvecs = [cb * L + lane for cb in range(C // L)]

    def fill_slab(ck, buf):
        @plsc.parallel_loop(0, N, 1, unroll=4)
        def pos_loop(p):
            pv = lane * 0 + (ck * N + p)
            bi = plsc.load_gather(idx_i, [pv]) * C
            bj = plsc.load_gather(idx_j, [pv]) * C
            for cb in range(C // L):
                v = plsc.load_gather(ti_v, [bi + cvecs[cb]])
                outbuf[buf, p, pl.ds(cb * L, L)] = v
                w = plsc.load_gather(tj_v, [bj + cvecs[cb]])
                outbuf[buf, p, pl.ds(C + cb * L, L)] = w

    def out_dma(ck, buf, sem):
        return pltpu.make_async_copy(
            outbuf.at[buf],
            out_hbm.at[b, pl.ds(p0 + ck * N, N), :],
            sem,
        )

    def ck_pair(s, _):
        for k in range(2):
            ck = 2 * s + k

            @pl.when(s >= 1)
            def _():
                out_dma(0, k, sems[k]).wait()

            fill_slab(ck, k)
            out_dma(ck, k, sems[k]).start()
        return 0

    lax.fori_loop(0, NCH // 2, ck_pair, 0)

    out_dma(0, 0, sem0).wait()
    out_dma(0, 1, sem1).wait()


@jax.jit
def _position_embedding_sc(i, j, table_i, table_j):
    mesh = plsc.VectorSubcoreMesh(core_axis_name="c", subcore_axis_name="s")
    fn = pl.kernel(
        _body,
        out_type=jax.ShapeDtypeStruct((B, NPOS, 2 * C), jnp.float32),
        mesh=mesh,
        scratch_types=[
            pltpu.VMEM((NROW * C,), jnp.float32),
            pltpu.VMEM((NROW * C,), jnp.float32),
            pltpu.VMEM((P,), jnp.int32),
            pltpu.VMEM((P,), jnp.int32),
            pltpu.VMEM((2, N, 2 * C), jnp.float32),
            pltpu.SemaphoreType.DMA,
            pltpu.SemaphoreType.DMA,
        ],
        compiler_params=pltpu.CompilerParams(needs_layout_passes=False),
    )
    outp = fn(i.reshape(-1), j.reshape(-1),
              table_i.reshape(-1), table_j.reshape(-1))
    return jnp.transpose(outp.reshape(B, H, W, 2 * C), (0, 3, 1, 2))


def kernel(i, j, table_i, table_j):
    return _position_embedding_sc(i, j, table_i, table_j)
